# baseline (speedup 1.0000x reference)
import functools

import jax
import jax.numpy as jnp
import numpy as np
from jax import lax
from jax.experimental import pallas as pl
from jax.experimental.pallas import tpu as pltpu
from jax.experimental.pallas import tpu_sc as plsc

N_NODES = 10000
D = 128
E = 320000
NB = 4
OUT_DIM = D * NB

NC = 2
NS = 16
NW = NC * NS
C = 80


def _tables_body(prop_ref, w1a_ref, w1b_ref, b1_ref, t1_ref, t2_ref):
    p = prop_ref[...]
    t1_ref[...] = (
        jnp.dot(p, w1a_ref[...], preferred_element_type=jnp.float32) + b1_ref[...]
    )
    t2_ref[...] = jnp.dot(p, w1b_ref[...], preferred_element_type=jnp.float32)


def _build_tables(prop, w1a, w1b, b1r):
    nblk = 5
    rows = N_NODES // nblk
    return pl.pallas_call(
        _tables_body,
        grid=(nblk,),
        in_specs=[
            pl.BlockSpec((rows, D), lambda i: (i, 0)),
            pl.BlockSpec((D, D), lambda i: (0, 0)),
            pl.BlockSpec((D, D), lambda i: (0, 0)),
            pl.BlockSpec((1, D), lambda i: (0, 0)),
        ],
        out_specs=[
            pl.BlockSpec((rows, D), lambda i: (i, 0)),
            pl.BlockSpec((rows, D), lambda i: (i, 0)),
        ],
        out_shape=[
            jax.ShapeDtypeStruct((N_NODES, D), jnp.float32),
            jax.ShapeDtypeStruct((N_NODES, D), jnp.float32),
        ],
    )(prop, w1a, w1b, b1r)


def _gather_add(t1, t2, idx_i, idx_j, es):
    epw = es // NW
    nchunk = epw // C
    mesh = plsc.VectorSubcoreMesh(core_axis_name="c", subcore_axis_name="s")

    @functools.partial(
        pl.kernel,
        mesh=mesh,
        out_type=jax.ShapeDtypeStruct((es, D), jnp.float32),
        scratch_types=[
            pltpu.VMEM((epw,), jnp.int32),
            pltpu.VMEM((epw,), jnp.int32),
            pltpu.VMEM((C, D), jnp.float32),
            pltpu.VMEM((C, D), jnp.float32),
            pltpu.VMEM((C, D), jnp.float32),
            pltpu.VMEM((C, D), jnp.float32),
            pltpu.SemaphoreType.DMA,
            pltpu.SemaphoreType.DMA,
        ],
    )
    def k(t1_hbm, t2_hbm, ii_hbm, jj_hbm, out_hbm,
          ii_v, jj_v, go0, go1, rb0, rb1, sg0, sg1):
        wid = lax.axis_index("s") * NC + lax.axis_index("c")
        base = wid * epw
        go = (go0, go1)
        rb = (rb0, rb1)
        sg = (sg0, sg1)

        pltpu.sync_copy(ii_hbm.at[pl.ds(base, epw)], ii_v)
        pltpu.sync_copy(jj_hbm.at[pl.ds(base, epw)], jj_v)

        def fire(t, b):
            sl = pl.ds(t * C, C)
            pltpu.make_async_copy(t1_hbm.at[ii_v.at[sl]], go[b], sg[b]).start()
            pltpu.make_async_copy(t2_hbm.at[jj_v.at[sl]], rb[b], sg[b]).start()

        def drain(b):
            pltpu.make_async_copy(t1_hbm.at[pl.ds(0, C)], go[b], sg[b]).wait()
            pltpu.make_async_copy(t2_hbm.at[pl.ds(0, C)], rb[b], sg[b]).wait()

        def process(t, b):
            drain(b)

            def row(r, c2):
                for v in range(D // 16):
                    s16 = pl.ds(v * 16, 16)
                    plsc.addupdate(go[b].at[r, s16], rb[b][r, s16])
                return c2

            lax.fori_loop(0, C, row, 0)
            pltpu.sync_copy(go[b], out_hbm.at[pl.ds(base + t * C, C)])

            @pl.when(t + 2 < nchunk)
            def _():
                fire(t + 2, b)

        fire(0, 0)
        fire(1, 1)

        def pair(i, carry):
            process(2 * i, 0)
            process(2 * i + 1, 1)
            return carry

        lax.fori_loop(0, nchunk // 2, pair, 0)
        if nchunk % 2:
            process(nchunk - 1, 0)

    return k(t1, t2, idx_i, idx_j)


def _edge_body(g_ref, bast_ref, w2_ref, *refs):
    out_ref = refs[-1]
    g = g_ref[...].astype(jnp.bfloat16)
    w2 = w2_ref[...]
    h0 = jnp.dot(g, w2[:, 0:D], preferred_element_type=jnp.float32)
    bas = jnp.transpose(bast_ref[...])
    acc = bas[:, 0:1] * h0
    for b in range(1, NB):
        acc = acc + bas[:, b : b + 1] * jnp.dot(
            g, w2[:, b * D : (b + 1) * D], preferred_element_type=jnp.float32
        )
    out_ref[...] = acc


def _edge_stage(g, basisT, w2p, acc, off_edges):
    es = g.shape[0]
    eb = 6400 if es % 6400 == 0 and off_edges % 6400 == 0 else 2560
    grid = es // eb
    off = off_edges // eb
    in_specs = [
        pl.BlockSpec((eb, D), lambda i: (i, 0)),
        pl.BlockSpec((NB, eb), lambda i, off=off: (0, off + i)),
        pl.BlockSpec((D, OUT_DIM), lambda i: (0, 0)),
    ]
    args = [g, basisT, w2p]
    aliases = {}
    if acc is not None:
        in_specs.append(pl.BlockSpec(memory_space=pl.ANY))
        args.append(acc)
        aliases = {3: 0}
    return pl.pallas_call(
        _edge_body,
        grid=(grid,),
        in_specs=in_specs,
        out_specs=pl.BlockSpec((eb, D), lambda i, off=off: (off + i, 0)),
        out_shape=jax.ShapeDtypeStruct((E, D), jnp.float32),
        input_output_aliases=aliases,
    )(*args)


def kernel(prop, idx_i, idx_j, basis, W1, b1, W2):
    w1a = W1[:D]
    w1b = W1[D:]
    b1r = b1.reshape(1, D)
    w2p = (
        W2.reshape(D, D, NB).transpose(0, 2, 1).reshape(D, OUT_DIM)
    ).astype(jnp.bfloat16)
    t1, t2 = _build_tables(prop, w1a, w1b, b1r)
    ii = idx_i.astype(jnp.int32)
    jj = idx_j.astype(jnp.int32)
    basisT = basis.T
    sizes = [25600] + [58880] * 5
    acc = None
    off = 0
    for es in sizes:
        sl = slice(off, off + es)
        g_s = _gather_add(t1, t2, ii[sl], jj[sl], es)
        acc = _edge_stage(g_s, basisT, w2p, acc, off)
        off += es
    return acc


# speedup vs baseline: 1.0244x; 1.0244x over previous
import functools

import jax
import jax.numpy as jnp
import numpy as np
from jax import lax
from jax.experimental import pallas as pl
from jax.experimental.pallas import tpu as pltpu
from jax.experimental.pallas import tpu_sc as plsc

N_NODES = 10000
D = 128
E = 320000
NB = 4
OUT_DIM = D * NB

NC = 2
NS = 16
NW = NC * NS
C = 80


def _tables_body(prop_ref, w1a_ref, w1b_ref, b1_ref, t1_ref, t2_ref):
    p = prop_ref[...]
    t1_ref[...] = (
        jnp.dot(p, w1a_ref[...], preferred_element_type=jnp.float32) + b1_ref[...]
    )
    t2_ref[...] = jnp.dot(p, w1b_ref[...], preferred_element_type=jnp.float32)


def _build_tables(prop, w1a, w1b, b1r):
    nblk = 5
    rows = N_NODES // nblk
    return pl.pallas_call(
        _tables_body,
        grid=(nblk,),
        in_specs=[
            pl.BlockSpec((rows, D), lambda i: (i, 0)),
            pl.BlockSpec((D, D), lambda i: (0, 0)),
            pl.BlockSpec((D, D), lambda i: (0, 0)),
            pl.BlockSpec((1, D), lambda i: (0, 0)),
        ],
        out_specs=[
            pl.BlockSpec((rows, D), lambda i: (i, 0)),
            pl.BlockSpec((rows, D), lambda i: (i, 0)),
        ],
        out_shape=[
            jax.ShapeDtypeStruct((N_NODES, D), jnp.float32),
            jax.ShapeDtypeStruct((N_NODES, D), jnp.float32),
        ],
    )(prop, w1a, w1b, b1r)


def _gather_add(t1, t2, idx_i, idx_j, es):
    epw = es // NW
    nchunk = epw // C
    mesh = plsc.VectorSubcoreMesh(core_axis_name="c", subcore_axis_name="s")

    @functools.partial(
        pl.kernel,
        mesh=mesh,
        out_type=jax.ShapeDtypeStruct((es, D), jnp.float32),
        scratch_types=[
            pltpu.VMEM((epw,), jnp.int32),
            pltpu.VMEM((epw,), jnp.int32),
            pltpu.VMEM((C, D), jnp.float32),
            pltpu.VMEM((C, D), jnp.float32),
            pltpu.VMEM((C, D), jnp.float32),
            pltpu.VMEM((C, D), jnp.float32),
            pltpu.SemaphoreType.DMA,
            pltpu.SemaphoreType.DMA,
        ],
    )
    def k(t1_hbm, t2_hbm, ii_hbm, jj_hbm, out_hbm,
          ii_v, jj_v, go0, go1, rb0, rb1, sg0, sg1):
        wid = lax.axis_index("s") * NC + lax.axis_index("c")
        base = wid * epw
        go = (go0, go1)
        rb = (rb0, rb1)
        sg = (sg0, sg1)

        pltpu.sync_copy(ii_hbm.at[pl.ds(base, epw)], ii_v)
        pltpu.sync_copy(jj_hbm.at[pl.ds(base, epw)], jj_v)

        def fire(t, b):
            sl = pl.ds(t * C, C)
            pltpu.make_async_copy(t1_hbm.at[ii_v.at[sl]], go[b], sg[b]).start()
            pltpu.make_async_copy(t2_hbm.at[jj_v.at[sl]], rb[b], sg[b]).start()

        def drain(b):
            pltpu.make_async_copy(t1_hbm.at[pl.ds(0, C)], go[b], sg[b]).wait()
            pltpu.make_async_copy(t2_hbm.at[pl.ds(0, C)], rb[b], sg[b]).wait()

        def process(t, b):
            drain(b)

            def row(r, c2):
                for v in range(D // 16):
                    s16 = pl.ds(v * 16, 16)
                    plsc.addupdate(go[b].at[r, s16], rb[b][r, s16])
                return c2

            lax.fori_loop(0, C, row, 0)
            pltpu.sync_copy(go[b], out_hbm.at[pl.ds(base + t * C, C)])

            @pl.when(t + 2 < nchunk)
            def _():
                fire(t + 2, b)

        fire(0, 0)
        fire(1, 1)

        def pair(i, carry):
            process(2 * i, 0)
            process(2 * i + 1, 1)
            return carry

        lax.fori_loop(0, nchunk // 2, pair, 0)
        if nchunk % 2:
            process(nchunk - 1, 0)

    return k(t1, t2, idx_i, idx_j)


def _edge_body(g_ref, bast_ref, w2_ref, *refs):
    out_ref = refs[-1]
    g = g_ref[...].astype(jnp.bfloat16)
    w2 = w2_ref[...]
    h0 = jnp.dot(g, w2[:, 0:D], preferred_element_type=jnp.float32)
    bas = jnp.transpose(bast_ref[...])
    acc = bas[:, 0:1] * h0
    for b in range(1, NB):
        acc = acc + bas[:, b : b + 1] * jnp.dot(
            g, w2[:, b * D : (b + 1) * D], preferred_element_type=jnp.float32
        )
    out_ref[...] = acc


def _edge_stage(g, basisT, w2p, acc, off_edges):
    es = g.shape[0]
    eb = 6400 if es % 6400 == 0 and off_edges % 6400 == 0 else 2560
    grid = es // eb
    off = off_edges // eb
    in_specs = [
        pl.BlockSpec((eb, D), lambda i: (i, 0)),
        pl.BlockSpec((NB, eb), lambda i, off=off: (0, off + i)),
        pl.BlockSpec((D, OUT_DIM), lambda i: (0, 0)),
    ]
    args = [g, basisT, w2p]
    aliases = {}
    if acc is not None:
        in_specs.append(pl.BlockSpec(memory_space=pl.ANY))
        args.append(acc)
        aliases = {3: 0}
    return pl.pallas_call(
        _edge_body,
        grid=(grid,),
        in_specs=in_specs,
        out_specs=pl.BlockSpec((eb, D), lambda i, off=off: (off + i, 0)),
        out_shape=jax.ShapeDtypeStruct((E, D), jnp.float32),
        input_output_aliases=aliases,
    )(*args)


def kernel(prop, idx_i, idx_j, basis, W1, b1, W2):
    w1a = W1[:D]
    w1b = W1[D:]
    b1r = b1.reshape(1, D)
    w2p = (
        W2.reshape(D, D, NB).transpose(0, 2, 1).reshape(D, OUT_DIM)
    ).astype(jnp.bfloat16)
    t1, t2 = _build_tables(prop, w1a, w1b, b1r)
    ii = idx_i.astype(jnp.int32)
    jj = idx_j.astype(jnp.int32)
    basisT = basis.T
    sizes = [25600] + [64000] * 4 + [38400]
    acc = None
    off = 0
    for es in sizes:
        sl = slice(off, off + es)
        g_s = _gather_add(t1, t2, ii[sl], jj[sl], es)
        acc = _edge_stage(g_s, basisT, w2p, acc, off)
        off += es
    return acc
